# baseline (speedup 1.0000x reference)
import jax
import jax.numpy as jnp
from jax import lax
from jax.experimental import pallas as pl
from jax.experimental.pallas import tpu as pltpu
from jax.experimental.pallas import tpu_sc as plsc

_ROWS = 1_000_000
_NUM_CORES = 2
_NUM_SUBCORES = 16
_NUM_WORKERS = _NUM_CORES * _NUM_SUBCORES
_ROWS_PER_W = (_ROWS // _NUM_WORKERS) // 8 * 8
_TAIL_BASE = _ROWS_PER_W * _NUM_WORKERS
_TAIL_ROWS = _ROWS - _TAIL_BASE


def _copy_body(user_hbm, item_hbm, out_u, out_i, sem_u, sem_i):
    wid = lax.axis_index("s") * _NUM_CORES + lax.axis_index("c")
    base = wid * _ROWS_PER_W
    sl = pl.ds(base, _ROWS_PER_W)
    cu = pltpu.make_async_copy(user_hbm.at[sl], out_u.at[sl], sem_u)
    ci = pltpu.make_async_copy(item_hbm.at[sl], out_i.at[sl], sem_i)
    cu.start()
    ci.start()
    cu.wait()
    ci.wait()

    @pl.when(wid == 0)
    def _tail():
        tl = pl.ds(_TAIL_BASE, _TAIL_ROWS)
        tu = pltpu.make_async_copy(user_hbm.at[tl], out_u.at[tl], sem_u)
        ti = pltpu.make_async_copy(item_hbm.at[tl], out_i.at[tl], sem_i)
        tu.start()
        ti.start()
        tu.wait()
        ti.wait()


def kernel(user_table, item_table):
    f = pl.kernel(
        _copy_body,
        out_type=(
            jax.ShapeDtypeStruct(user_table.shape, user_table.dtype),
            jax.ShapeDtypeStruct(item_table.shape, item_table.dtype),
        ),
        mesh=plsc.VectorSubcoreMesh(core_axis_name="c", subcore_axis_name="s"),
        scratch_types=[pltpu.SemaphoreType.DMA, pltpu.SemaphoreType.DMA],
    )
    return f(user_table, item_table)


# speedup vs baseline: 5.9035x; 5.9035x over previous
import jax
import jax.numpy as jnp
from jax import lax
from jax.experimental import pallas as pl
from jax.experimental.pallas import tpu as pltpu
from jax.experimental.pallas import tpu_sc as plsc

_ROWS = 1_000_000
_DIM = 16
_VROWS = _ROWS * _DIM // 128
_NUM_CORES = 2
_NUM_SUBCORES = 16
_NUM_WORKERS = _NUM_CORES * _NUM_SUBCORES
_ROWS_PER_W = (_VROWS // _NUM_WORKERS) // 8 * 8
_TAIL_BASE = _ROWS_PER_W * _NUM_WORKERS
_TAIL_ROWS = _VROWS - _TAIL_BASE


def _copy_body(user_hbm, item_hbm, out_u, out_i, sem_u, sem_i):
    wid = lax.axis_index("s") * _NUM_CORES + lax.axis_index("c")
    base = wid * _ROWS_PER_W
    sl = pl.ds(base, _ROWS_PER_W)
    cu = pltpu.make_async_copy(user_hbm.at[sl], out_u.at[sl], sem_u)
    ci = pltpu.make_async_copy(item_hbm.at[sl], out_i.at[sl], sem_i)
    cu.start()
    ci.start()
    cu.wait()
    ci.wait()

    @pl.when(wid == 0)
    def _tail():
        tl = pl.ds(_TAIL_BASE, _TAIL_ROWS)
        tu = pltpu.make_async_copy(user_hbm.at[tl], out_u.at[tl], sem_u)
        ti = pltpu.make_async_copy(item_hbm.at[tl], out_i.at[tl], sem_i)
        tu.start()
        ti.start()
        tu.wait()
        ti.wait()


def kernel(user_table, item_table):
    f = pl.kernel(
        _copy_body,
        out_type=(
            jax.ShapeDtypeStruct((_VROWS, 128), user_table.dtype),
            jax.ShapeDtypeStruct((_VROWS, 128), item_table.dtype),
        ),
        mesh=plsc.VectorSubcoreMesh(core_axis_name="c", subcore_axis_name="s"),
        scratch_types=[pltpu.SemaphoreType.DMA, pltpu.SemaphoreType.DMA],
    )
    out_u, out_i = f(
        user_table.reshape(_VROWS, 128), item_table.reshape(_VROWS, 128)
    )
    return (
        out_u.reshape(_ROWS, _DIM),
        out_i.reshape(_ROWS, _DIM),
    )
